# baseline (speedup 1.0000x reference)
import functools

import jax
import jax.numpy as jnp
from jax import lax
from jax.experimental import pallas as pl
from jax.experimental.pallas import tpu as pltpu
from jax.experimental.pallas import tpu_sc as plsc

N = 10000
E = 320000
EPS = 1e-5

NC = 2
NS = 16
BLK = 128

N_PAD = 10240
ROWS_PER_TILE = N_PAD // NS
E_PAD = 80 * NC * NS * BLK
NBLK_TOT = E_PAD // BLK
NBLK16 = NBLK_TOT // NS
NBLK32 = NBLK_TOT // (NC * NS)
DUMMY = N

BM = 512

_mesh = plsc.VectorSubcoreMesh(core_axis_name="c", subcore_axis_name="s",
                               num_cores=NC, num_subcores=NS)


def _f32(shape):
    return jax.ShapeDtypeStruct(shape, jnp.float32)


@functools.partial(
    pl.kernel,
    out_type=_f32((N_PAD, 16)),
    mesh=_mesh,
    scratch_types=[
        pltpu.VMEM_SHARED((N_PAD, 16), jnp.float32),
        pltpu.VMEM((BLK, 16), jnp.float32),
        pltpu.VMEM((BLK, 16), jnp.float32),
        pltpu.VMEM((BLK,), jnp.int32),
        pltpu.VMEM((BLK,), jnp.int32),
    ],
)
def _deg_kernel(dst_hbm, deg_out, acc, zbuf, obuf, da, db):
    c = lax.axis_index("c")
    s = lax.axis_index("s")
    e0 = s * NBLK16 * BLK

    def fill(i, _):
        zbuf[i] = jnp.zeros((16,), jnp.float32)
        obuf[i] = jnp.ones((16,), jnp.float32)
        return 0

    lax.fori_loop(0, BLK, fill, 0, unroll=False)

    def zero_chunk(k, _):
        pltpu.sync_copy(zbuf, acc.at[pl.ds(s * ROWS_PER_TILE + k * BLK, BLK)])
        return 0

    lax.fori_loop(0, ROWS_PER_TILE // BLK, zero_chunk, 0, unroll=False)
    plsc.subcore_barrier()

    def pair(j, _):
        base = e0 + 2 * j * BLK
        pltpu.sync_copy(dst_hbm.at[pl.ds(base, BLK)], da)
        pltpu.sync_copy(dst_hbm.at[pl.ds(base + BLK, BLK)], db)
        pltpu.sync_copy(obuf, acc.at[da], add=True)
        pltpu.sync_copy(obuf, acc.at[db], add=True)
        return 0

    lax.fori_loop(0, NBLK16 // 2, pair, 0, unroll=False)
    plsc.subcore_barrier()

    @pl.when(c == 0)
    def _():
        pltpu.sync_copy(
            acc.at[pl.ds(s * ROWS_PER_TILE, ROWS_PER_TILE)],
            deg_out.at[pl.ds(s * ROWS_PER_TILE, ROWS_PER_TILE)],
        )


@functools.partial(
    pl.kernel,
    out_type=(_f32((N_PAD, 128)), _f32((N_PAD, 128))),
    mesh=_mesh,
    scratch_types=[
        pltpu.VMEM_SHARED((N_PAD, 128), jnp.float32),
        pltpu.VMEM((BLK,), jnp.int32),
        pltpu.VMEM((BLK,), jnp.int32),
        pltpu.VMEM((BLK,), jnp.int32),
        pltpu.VMEM((BLK,), jnp.int32),
        pltpu.VMEM((BLK, 128), jnp.float32),
        pltpu.VMEM((BLK, 128), jnp.float32),
        pltpu.SemaphoreType.DMA,
        pltpu.SemaphoreType.DMA,
    ],
)
def _spmm_fsplit(u0_hbm, u1_hbm, src_hbm, dst_hbm, a0_out, a1_out,
                 acc, sa, sb, da, db, rows_a, rows_b, sem_a, sem_b):
    c = lax.axis_index("c")
    s = lax.axis_index("s")
    r0 = s * ROWS_PER_TILE
    e0 = s * NBLK16 * BLK

    @pl.when(c == 0)
    def _():
        pltpu.sync_copy(u0_hbm.at[pl.ds(r0, ROWS_PER_TILE)],
                        acc.at[pl.ds(r0, ROWS_PER_TILE)])

    @pl.when(c == 1)
    def _():
        pltpu.sync_copy(u1_hbm.at[pl.ds(r0, ROWS_PER_TILE)],
                        acc.at[pl.ds(r0, ROWS_PER_TILE)])

    plsc.subcore_barrier()

    def pair_for(u_hbm):
        def pair(j, _):
            base = e0 + 2 * j * BLK
            pltpu.sync_copy(src_hbm.at[pl.ds(base, BLK)], sa)
            pltpu.sync_copy(src_hbm.at[pl.ds(base + BLK, BLK)], sb)
            pltpu.sync_copy(dst_hbm.at[pl.ds(base, BLK)], da)
            pltpu.sync_copy(dst_hbm.at[pl.ds(base + BLK, BLK)], db)
            ga = pltpu.async_copy(u_hbm.at[sa], rows_a, sem_a)
            gb = pltpu.async_copy(u_hbm.at[sb], rows_b, sem_b)
            ga.wait()
            pltpu.sync_copy(rows_a, acc.at[da], add=True)
            gb.wait()
            pltpu.sync_copy(rows_b, acc.at[db], add=True)
            return 0
        return pair

    @pl.when(c == 0)
    def _():
        lax.fori_loop(0, NBLK16 // 2, pair_for(u0_hbm), 0, unroll=False)

    @pl.when(c == 1)
    def _():
        lax.fori_loop(0, NBLK16 // 2, pair_for(u1_hbm), 0, unroll=False)

    plsc.subcore_barrier()

    @pl.when(c == 0)
    def _():
        pltpu.sync_copy(acc.at[pl.ds(r0, ROWS_PER_TILE)],
                        a0_out.at[pl.ds(r0, ROWS_PER_TILE)])

    @pl.when(c == 1)
    def _():
        pltpu.sync_copy(acc.at[pl.ds(r0, ROWS_PER_TILE)],
                        a1_out.at[pl.ds(r0, ROWS_PER_TILE)])


@functools.partial(
    pl.kernel,
    out_type=(_f32((N_PAD, 128)), _f32((N_PAD, 128))),
    mesh=_mesh,
    scratch_types=[
        pltpu.VMEM_SHARED((N_PAD, 128), jnp.float32),
        pltpu.VMEM((BLK,), jnp.int32),
        pltpu.VMEM((BLK,), jnp.int32),
        pltpu.VMEM((BLK,), jnp.int32),
        pltpu.VMEM((BLK,), jnp.int32),
        pltpu.VMEM((BLK, 128), jnp.float32),
        pltpu.VMEM((BLK, 128), jnp.float32),
        pltpu.SemaphoreType.DMA,
        pltpu.SemaphoreType.DMA,
    ],
)
def _spmm_esplit(u_hbm, src_hbm, dst_hbm, aa_out, ab_out,
                 acc, sa, sb, da, db, rows_a, rows_b, sem_a, sem_b):
    c = lax.axis_index("c")
    s = lax.axis_index("s")
    r0 = s * ROWS_PER_TILE
    e0 = (c * NS + s) * NBLK32 * BLK

    pltpu.sync_copy(u_hbm.at[pl.ds(r0, ROWS_PER_TILE)],
                    acc.at[pl.ds(r0, ROWS_PER_TILE)])
    plsc.subcore_barrier()

    def pair(j, _):
        base = e0 + 2 * j * BLK
        pltpu.sync_copy(src_hbm.at[pl.ds(base, BLK)], sa)
        pltpu.sync_copy(src_hbm.at[pl.ds(base + BLK, BLK)], sb)
        pltpu.sync_copy(dst_hbm.at[pl.ds(base, BLK)], da)
        pltpu.sync_copy(dst_hbm.at[pl.ds(base + BLK, BLK)], db)
        ga = pltpu.async_copy(u_hbm.at[sa], rows_a, sem_a)
        gb = pltpu.async_copy(u_hbm.at[sb], rows_b, sem_b)
        ga.wait()
        pltpu.sync_copy(rows_a, acc.at[da], add=True)
        gb.wait()
        pltpu.sync_copy(rows_b, acc.at[db], add=True)
        return 0

    lax.fori_loop(0, NBLK32 // 2, pair, 0, unroll=False)
    plsc.subcore_barrier()

    @pl.when(c == 0)
    def _():
        pltpu.sync_copy(acc.at[pl.ds(r0, ROWS_PER_TILE)],
                        aa_out.at[pl.ds(r0, ROWS_PER_TILE)])

    @pl.when(c == 1)
    def _():
        pltpu.sync_copy(acc.at[pl.ds(r0, ROWS_PER_TILE)],
                        ab_out.at[pl.ds(r0, ROWS_PER_TILE)])


def _row_spec(width):
    return pl.BlockSpec((BM, width), lambda i: (i, 0))


def _full_spec(shape):
    return pl.BlockSpec(shape, lambda i: tuple(0 for _ in shape))


def _l1_body(x_ref, w_ref, g_ref, v_ref, deg_ref, u0_ref, u1_ref, dinv_ref):
    i = pl.program_id(0)
    rows = i * BM + lax.broadcasted_iota(jnp.int32, (BM, 1), 0)
    deg = deg_ref[:, 0:1] + 1.0
    dinv = jnp.where(rows < N, lax.rsqrt(deg), 0.0)
    dinv_ref[...] = dinv
    s = g_ref[...] * lax.rsqrt(v_ref[...] + EPS)
    h = jnp.dot(x_ref[...], w_ref[...], preferred_element_type=jnp.float32)
    u = h * s * dinv
    u0_ref[...] = u[:, :128]
    u1_ref[...] = u[:, 128:]


def _mid_body(a0_ref, a1_ref, dinv_ref, b_ref, g_ref, beta_ref, m_ref, v_ref,
              gn_ref, vn_ref, w_ref, u0_ref, u1_ref):
    sp = g_ref[...] * lax.rsqrt(v_ref[...] + EPS)
    bias = b_ref[...] * sp + beta_ref[...] - m_ref[...] * sp
    dinv = dinv_ref[...]
    acc = jnp.concatenate([a0_ref[...], a1_ref[...]], axis=1)
    t = jnp.maximum(acc * dinv + bias, 0.0)
    sn = gn_ref[...] * lax.rsqrt(vn_ref[...] + EPS)
    u = jnp.dot(t, w_ref[...], preferred_element_type=jnp.float32) * sn * dinv
    u0_ref[...] = u[:, :128]
    u1_ref[...] = u[:, 128:]


def _l3_body(a0_ref, a1_ref, dinv_ref, b_ref, g_ref, beta_ref, m_ref, v_ref,
             w_ref, u_ref):
    sp = g_ref[...] * lax.rsqrt(v_ref[...] + EPS)
    bias = b_ref[...] * sp + beta_ref[...] - m_ref[...] * sp
    dinv = dinv_ref[...]
    acc = jnp.concatenate([a0_ref[...], a1_ref[...]], axis=1)
    t = jnp.maximum(acc * dinv + bias, 0.0)
    u_ref[...] = jnp.dot(t, w_ref[...],
                         preferred_element_type=jnp.float32) * dinv


def _fin_body(aa_ref, ab_ref, u_ref, dinv_ref, b_ref, o_ref):
    acc = aa_ref[...] + ab_ref[...] - u_ref[...]
    o_ref[...] = acc * dinv_ref[...] + b_ref[...]


_GRID = (N_PAD // BM,)

_l1_call = pl.pallas_call(
    _l1_body,
    grid=_GRID,
    in_specs=[_row_spec(128), _full_spec((128, 256)), _full_spec((1, 256)),
              _full_spec((1, 256)), _row_spec(16)],
    out_specs=[_row_spec(128), _row_spec(128), _row_spec(1)],
    out_shape=[_f32((N_PAD, 128)), _f32((N_PAD, 128)), _f32((N_PAD, 1))],
)

_mid_call = pl.pallas_call(
    _mid_body,
    grid=_GRID,
    in_specs=[_row_spec(128), _row_spec(128), _row_spec(1)]
             + [_full_spec((1, 256))] * 7
             + [_full_spec((256, 256))],
    out_specs=[_row_spec(128), _row_spec(128)],
    out_shape=[_f32((N_PAD, 128)), _f32((N_PAD, 128))],
)

_l3_call = pl.pallas_call(
    _l3_body,
    grid=_GRID,
    in_specs=[_row_spec(128), _row_spec(128), _row_spec(1)]
             + [_full_spec((1, 256))] * 5
             + [_full_spec((256, 128))],
    out_specs=_row_spec(128),
    out_shape=_f32((N_PAD, 128)),
)

_fin_call = pl.pallas_call(
    _fin_body,
    grid=_GRID,
    in_specs=[_row_spec(128), _row_spec(128), _row_spec(128), _row_spec(1),
              _full_spec((1, 128))],
    out_specs=_row_spec(128),
    out_shape=_f32((N_PAD, 128)),
)


@jax.jit
def kernel(x, edge_index, W1, b1, W2, b2, W3, b3,
           g1, beta1, m1, v1, g2, beta2, m2, v2):
    src = edge_index[0].astype(jnp.int32)
    dst = edge_index[1].astype(jnp.int32)
    pad = jnp.full((E_PAD - E,), DUMMY, jnp.int32)
    src_p = jnp.concatenate([src, pad])
    dst_p = jnp.concatenate([dst, pad])
    x_p = jnp.zeros((N_PAD, 128), jnp.float32).at[:N].set(x)

    row = lambda a: a.reshape(1, -1)

    deg = _deg_kernel(dst_p)

    u1_0, u1_1, dinv = _l1_call(x_p, W1, row(g1), row(v1), deg)
    a1_0, a1_1 = _spmm_fsplit(u1_0, u1_1, src_p, dst_p)

    u2_0, u2_1 = _mid_call(a1_0, a1_1, dinv, row(b1), row(g1), row(beta1),
                           row(m1), row(v1), row(g2), row(v2), W2)
    a2_0, a2_1 = _spmm_fsplit(u2_0, u2_1, src_p, dst_p)

    u3 = _l3_call(a2_0, a2_1, dinv, row(b2), row(g2), row(beta2),
                  row(m2), row(v2), W3)
    a3a, a3b = _spmm_esplit(u3, src_p, dst_p)

    out = _fin_call(a3a, a3b, u3, dinv, row(b3))
    return out[:N]


# speedup vs baseline: 1.3675x; 1.3675x over previous
import functools

import jax
import jax.numpy as jnp
from jax import lax
from jax.experimental import pallas as pl
from jax.experimental.pallas import tpu as pltpu
from jax.experimental.pallas import tpu_sc as plsc

N = 10000
E = 320000
EPS = 1e-5

NC = 2
NS = 16
BLK = 128

N_PAD = 10240
ROWS_PER_TILE = N_PAD // NS
E_PAD = 79 * NC * NS * BLK
DUMMY = N

BM = 512

_mesh = plsc.VectorSubcoreMesh(core_axis_name="c", subcore_axis_name="s",
                               num_cores=NC, num_subcores=NS)


def _f32(shape):
    return jax.ShapeDtypeStruct(shape, jnp.float32)


@functools.partial(
    pl.kernel,
    out_type=(_f32((N_PAD, 16)), _f32((N_PAD, 16))),
    mesh=_mesh,
    scratch_types=[
        pltpu.VMEM_SHARED((N_PAD, 16), jnp.float32),
        pltpu.VMEM((BLK, 16), jnp.float32),
        pltpu.VMEM((BLK, 16), jnp.float32),
        pltpu.VMEM((BLK,), jnp.int32),
    ],
)
def _deg_kernel(dst_hbm, deg0_out, deg1_out, acc, zbuf, obuf, dst_v):
    c = lax.axis_index("c")
    s = lax.axis_index("s")

    def fill(i, _):
        zbuf[i] = jnp.zeros((16,), jnp.float32)
        obuf[i] = jnp.ones((16,), jnp.float32)
        return 0

    lax.fori_loop(0, BLK, fill, 0, unroll=False)

    def zero_chunk(k, _):
        pltpu.sync_copy(zbuf, acc.at[pl.ds(s * ROWS_PER_TILE + k * BLK, BLK)])
        return 0

    lax.fori_loop(0, ROWS_PER_TILE // BLK, zero_chunk, 0, unroll=False)
    plsc.subcore_barrier()

    nblk = E_PAD // (NC * NS * BLK)

    def step(i, _):
        base = (c * NS + s) * nblk * BLK + i * BLK
        pltpu.sync_copy(dst_hbm.at[pl.ds(base, BLK)], dst_v)
        pltpu.sync_copy(obuf, acc.at[dst_v], add=True)
        return 0

    lax.fori_loop(0, nblk, step, 0, unroll=False)
    plsc.subcore_barrier()

    @pl.when(c == 0)
    def _():
        pltpu.sync_copy(
            acc.at[pl.ds(s * ROWS_PER_TILE, ROWS_PER_TILE)],
            deg0_out.at[pl.ds(s * ROWS_PER_TILE, ROWS_PER_TILE)],
        )

    @pl.when(c == 1)
    def _():
        pltpu.sync_copy(
            acc.at[pl.ds(s * ROWS_PER_TILE, ROWS_PER_TILE)],
            deg1_out.at[pl.ds(s * ROWS_PER_TILE, ROWS_PER_TILE)],
        )


@functools.partial(
    pl.kernel,
    out_type=(_f32((N_PAD, 128)), _f32((N_PAD, 128))),
    mesh=_mesh,
    scratch_types=[
        pltpu.VMEM_SHARED((N_PAD, 128), jnp.float32),
        pltpu.VMEM((BLK,), jnp.int32),
        pltpu.VMEM((BLK,), jnp.int32),
        pltpu.VMEM((BLK, 128), jnp.float32),
        pltpu.SemaphoreType.DMA,
    ],
)
def _spmm_fsplit(u0_hbm, u1_hbm, src_hbm, dst_hbm, a0_out, a1_out,
                 acc, src_v, dst_v, rows_v, sem):
    c = lax.axis_index("c")
    s = lax.axis_index("s")
    r0 = s * ROWS_PER_TILE

    @pl.when(c == 0)
    def _():
        pltpu.sync_copy(u0_hbm.at[pl.ds(r0, ROWS_PER_TILE)],
                        acc.at[pl.ds(r0, ROWS_PER_TILE)])

    @pl.when(c == 1)
    def _():
        pltpu.sync_copy(u1_hbm.at[pl.ds(r0, ROWS_PER_TILE)],
                        acc.at[pl.ds(r0, ROWS_PER_TILE)])

    plsc.subcore_barrier()

    nblk = E_PAD // (NS * BLK)

    def step(i, _):
        base = s * nblk * BLK + i * BLK
        pltpu.sync_copy(src_hbm.at[pl.ds(base, BLK)], src_v)
        pltpu.sync_copy(dst_hbm.at[pl.ds(base, BLK)], dst_v)

        @pl.when(c == 0)
        def _():
            pltpu.async_copy(u0_hbm.at[src_v], rows_v, sem).wait()

        @pl.when(c == 1)
        def _():
            pltpu.async_copy(u1_hbm.at[src_v], rows_v, sem).wait()

        pltpu.sync_copy(rows_v, acc.at[dst_v], add=True)
        return 0

    lax.fori_loop(0, nblk, step, 0, unroll=False)
    plsc.subcore_barrier()

    @pl.when(c == 0)
    def _():
        pltpu.sync_copy(acc.at[pl.ds(r0, ROWS_PER_TILE)],
                        a0_out.at[pl.ds(r0, ROWS_PER_TILE)])

    @pl.when(c == 1)
    def _():
        pltpu.sync_copy(acc.at[pl.ds(r0, ROWS_PER_TILE)],
                        a1_out.at[pl.ds(r0, ROWS_PER_TILE)])


@functools.partial(
    pl.kernel,
    out_type=(_f32((N_PAD, 128)), _f32((N_PAD, 128))),
    mesh=_mesh,
    scratch_types=[
        pltpu.VMEM_SHARED((N_PAD, 128), jnp.float32),
        pltpu.VMEM((BLK,), jnp.int32),
        pltpu.VMEM((BLK,), jnp.int32),
        pltpu.VMEM((BLK, 128), jnp.float32),
        pltpu.SemaphoreType.DMA,
    ],
)
def _spmm_esplit(u_hbm, src_hbm, dst_hbm, aa_out, ab_out,
                 acc, src_v, dst_v, rows_v, sem):
    c = lax.axis_index("c")
    s = lax.axis_index("s")
    r0 = s * ROWS_PER_TILE

    pltpu.sync_copy(u_hbm.at[pl.ds(r0, ROWS_PER_TILE)],
                    acc.at[pl.ds(r0, ROWS_PER_TILE)])
    plsc.subcore_barrier()

    nblk = E_PAD // (NC * NS * BLK)

    def step(i, _):
        base = (c * NS + s) * nblk * BLK + i * BLK
        pltpu.sync_copy(src_hbm.at[pl.ds(base, BLK)], src_v)
        pltpu.sync_copy(dst_hbm.at[pl.ds(base, BLK)], dst_v)
        pltpu.async_copy(u_hbm.at[src_v], rows_v, sem).wait()
        pltpu.sync_copy(rows_v, acc.at[dst_v], add=True)
        return 0

    lax.fori_loop(0, nblk, step, 0, unroll=False)
    plsc.subcore_barrier()

    @pl.when(c == 0)
    def _():
        pltpu.sync_copy(acc.at[pl.ds(r0, ROWS_PER_TILE)],
                        aa_out.at[pl.ds(r0, ROWS_PER_TILE)])

    @pl.when(c == 1)
    def _():
        pltpu.sync_copy(acc.at[pl.ds(r0, ROWS_PER_TILE)],
                        ab_out.at[pl.ds(r0, ROWS_PER_TILE)])


def _row_spec(width):
    return pl.BlockSpec((BM, width), lambda i: (i, 0))


def _full_spec(shape):
    return pl.BlockSpec(shape, lambda i: tuple(0 for _ in shape))


def _l1_body(x_ref, w_ref, g_ref, v_ref, deg0_ref, deg1_ref,
             u0_ref, u1_ref, dinv_ref):
    i = pl.program_id(0)
    rows = i * BM + lax.broadcasted_iota(jnp.int32, (BM, 1), 0)
    deg = deg0_ref[:, 0:1] + deg1_ref[:, 0:1] + 1.0
    dinv = jnp.where(rows < N, lax.rsqrt(deg), 0.0)
    dinv_ref[...] = dinv
    s = g_ref[...] * lax.rsqrt(v_ref[...] + EPS)
    h = jnp.dot(x_ref[...], w_ref[...], preferred_element_type=jnp.float32)
    u = h * s * dinv
    u0_ref[...] = u[:, :128]
    u1_ref[...] = u[:, 128:]


def _mid_body(a0_ref, a1_ref, dinv_ref, b_ref, g_ref, beta_ref, m_ref, v_ref,
              gn_ref, vn_ref, w_ref, u0_ref, u1_ref):
    sp = g_ref[...] * lax.rsqrt(v_ref[...] + EPS)
    bias = b_ref[...] * sp + beta_ref[...] - m_ref[...] * sp
    dinv = dinv_ref[...]
    acc = jnp.concatenate([a0_ref[...], a1_ref[...]], axis=1)
    t = jnp.maximum(acc * dinv + bias, 0.0)
    sn = gn_ref[...] * lax.rsqrt(vn_ref[...] + EPS)
    u = jnp.dot(t, w_ref[...], preferred_element_type=jnp.float32) * sn * dinv
    u0_ref[...] = u[:, :128]
    u1_ref[...] = u[:, 128:]


def _l3_body(a0_ref, a1_ref, dinv_ref, b_ref, g_ref, beta_ref, m_ref, v_ref,
             w_ref, u_ref):
    sp = g_ref[...] * lax.rsqrt(v_ref[...] + EPS)
    bias = b_ref[...] * sp + beta_ref[...] - m_ref[...] * sp
    dinv = dinv_ref[...]
    acc = jnp.concatenate([a0_ref[...], a1_ref[...]], axis=1)
    t = jnp.maximum(acc * dinv + bias, 0.0)
    u_ref[...] = jnp.dot(t, w_ref[...],
                         preferred_element_type=jnp.float32) * dinv


def _fin_body(aa_ref, ab_ref, u_ref, dinv_ref, b_ref, o_ref):
    acc = aa_ref[...] + ab_ref[...] - u_ref[...]
    o_ref[...] = acc * dinv_ref[...] + b_ref[...]


_GRID = (N_PAD // BM,)

_l1_call = pl.pallas_call(
    _l1_body,
    grid=_GRID,
    in_specs=[_row_spec(128), _full_spec((128, 256)), _full_spec((1, 256)),
              _full_spec((1, 256)), _row_spec(16), _row_spec(16)],
    out_specs=[_row_spec(128), _row_spec(128), _row_spec(1)],
    out_shape=[_f32((N_PAD, 128)), _f32((N_PAD, 128)), _f32((N_PAD, 1))],
)

_mid_call = pl.pallas_call(
    _mid_body,
    grid=_GRID,
    in_specs=[_row_spec(128), _row_spec(128), _row_spec(1)]
             + [_full_spec((1, 256))] * 7
             + [_full_spec((256, 256))],
    out_specs=[_row_spec(128), _row_spec(128)],
    out_shape=[_f32((N_PAD, 128)), _f32((N_PAD, 128))],
)

_l3_call = pl.pallas_call(
    _l3_body,
    grid=_GRID,
    in_specs=[_row_spec(128), _row_spec(128), _row_spec(1)]
             + [_full_spec((1, 256))] * 5
             + [_full_spec((256, 128))],
    out_specs=_row_spec(128),
    out_shape=_f32((N_PAD, 128)),
)

_fin_call = pl.pallas_call(
    _fin_body,
    grid=_GRID,
    in_specs=[_row_spec(128), _row_spec(128), _row_spec(128), _row_spec(1),
              _full_spec((1, 128))],
    out_specs=_row_spec(128),
    out_shape=_f32((N_PAD, 128)),
)


@jax.jit
def kernel(x, edge_index, W1, b1, W2, b2, W3, b3,
           g1, beta1, m1, v1, g2, beta2, m2, v2):
    src = edge_index[0].astype(jnp.int32)
    dst = edge_index[1].astype(jnp.int32)
    pad = jnp.full((E_PAD - E,), DUMMY, jnp.int32)
    src_p = jnp.concatenate([src, pad])
    dst_p = jnp.concatenate([dst, pad])
    x_p = jnp.zeros((N_PAD, 128), jnp.float32).at[:N].set(x)

    row = lambda a: a.reshape(1, -1)

    deg0, deg1 = _deg_kernel(dst_p)

    u1_0, u1_1, dinv = _l1_call(x_p, W1, row(g1), row(v1), deg0, deg1)
    a1_0, a1_1 = _spmm_fsplit(u1_0, u1_1, src_p, dst_p)

    u2_0, u2_1 = _mid_call(a1_0, a1_1, dinv, row(b1), row(g1), row(beta1),
                           row(m1), row(v1), row(g2), row(v2), W2)
    a2_0, a2_1 = _spmm_fsplit(u2_0, u2_1, src_p, dst_p)

    u3 = _l3_call(a2_0, a2_1, dinv, row(b2), row(g2), row(beta2),
                  row(m2), row(v2), W3)
    a3a, a3b = _spmm_esplit(u3, src_p, dst_p)

    out = _fin_call(a3a, a3b, u3, dinv, row(b3))
    return out[:N]


# speedup vs baseline: 1.6821x; 1.2300x over previous
import functools

import jax
import jax.numpy as jnp
from jax import lax
from jax.experimental import pallas as pl
from jax.experimental.pallas import tpu as pltpu
from jax.experimental.pallas import tpu_sc as plsc

N = 10000
E = 320000
EPS = 1e-5

NC = 2
NS = 16
BLK = 128

N_PAD = 10240
ROWS_PER_TILE = N_PAD // NS
E_PAD = 79 * NC * NS * BLK
DUMMY = N

BM = 512

_mesh = plsc.VectorSubcoreMesh(core_axis_name="c", subcore_axis_name="s",
                               num_cores=NC, num_subcores=NS)


def _f32(shape):
    return jax.ShapeDtypeStruct(shape, jnp.float32)


@functools.partial(
    pl.kernel,
    out_type=(_f32((N_PAD, 16)), _f32((N_PAD, 16))),
    mesh=_mesh,
    scratch_types=[
        pltpu.VMEM_SHARED((N_PAD, 16), jnp.float32),
        pltpu.VMEM((BLK, 16), jnp.float32),
        pltpu.VMEM((BLK, 16), jnp.float32),
        pltpu.VMEM((BLK,), jnp.int32),
        pltpu.VMEM((BLK,), jnp.int32),
        pltpu.SemaphoreType.DMA,
        pltpu.SemaphoreType.DMA,
    ],
)
def _deg_kernel(dst_hbm, deg0_out, deg1_out, acc, zbuf, obuf, da0, da1,
                isem0, isem1):
    c = lax.axis_index("c")
    s = lax.axis_index("s")

    def fill(i, _):
        zbuf[i] = jnp.zeros((16,), jnp.float32)
        obuf[i] = jnp.ones((16,), jnp.float32)
        return 0

    lax.fori_loop(0, BLK, fill, 0, unroll=False)

    def zero_chunk(k, _):
        pltpu.sync_copy(zbuf, acc.at[pl.ds(s * ROWS_PER_TILE + k * BLK, BLK)])
        return 0

    lax.fori_loop(0, ROWS_PER_TILE // BLK, zero_chunk, 0, unroll=False)
    plsc.subcore_barrier()

    nblk = E_PAD // (NC * NS * BLK)
    e0 = (c * NS + s) * nblk * BLK

    pltpu.async_copy(dst_hbm.at[pl.ds(e0, BLK)], da0, isem0)
    pltpu.async_copy(dst_hbm.at[pl.ds(e0 + BLK, BLK)], da1, isem1)

    def pair(j, _):
        b_even = e0 + 2 * j * BLK
        pltpu.make_async_copy(dst_hbm.at[pl.ds(b_even, BLK)], da0,
                              isem0).wait()
        pltpu.sync_copy(obuf, acc.at[da0], add=True)

        @pl.when(2 * j + 2 < nblk)
        def _():
            pltpu.async_copy(dst_hbm.at[pl.ds(b_even + 2 * BLK, BLK)], da0,
                             isem0)

        pltpu.make_async_copy(dst_hbm.at[pl.ds(b_even + BLK, BLK)], da1,
                              isem1).wait()
        pltpu.sync_copy(obuf, acc.at[da1], add=True)

        @pl.when(2 * j + 3 < nblk)
        def _():
            pltpu.async_copy(dst_hbm.at[pl.ds(b_even + 3 * BLK, BLK)], da1,
                             isem1)

        return 0

    lax.fori_loop(0, nblk // 2, pair, 0, unroll=False)
    if nblk % 2:
        b_t = e0 + (nblk - 1) * BLK
        pltpu.make_async_copy(dst_hbm.at[pl.ds(b_t, BLK)], da0, isem0).wait()
        pltpu.sync_copy(obuf, acc.at[da0], add=True)
    plsc.subcore_barrier()

    @pl.when(c == 0)
    def _():
        pltpu.sync_copy(
            acc.at[pl.ds(s * ROWS_PER_TILE, ROWS_PER_TILE)],
            deg0_out.at[pl.ds(s * ROWS_PER_TILE, ROWS_PER_TILE)],
        )

    @pl.when(c == 1)
    def _():
        pltpu.sync_copy(
            acc.at[pl.ds(s * ROWS_PER_TILE, ROWS_PER_TILE)],
            deg1_out.at[pl.ds(s * ROWS_PER_TILE, ROWS_PER_TILE)],
        )


@functools.partial(
    pl.kernel,
    out_type=(_f32((N_PAD, 128)), _f32((N_PAD, 128))),
    mesh=_mesh,
    scratch_types=[
        pltpu.VMEM_SHARED((N_PAD, 128), jnp.float32),
        pltpu.VMEM((BLK,), jnp.int32),
        pltpu.VMEM((BLK,), jnp.int32),
        pltpu.VMEM((BLK,), jnp.int32),
        pltpu.VMEM((BLK,), jnp.int32),
        pltpu.VMEM((BLK, 128), jnp.float32),
        pltpu.SemaphoreType.DMA,
        pltpu.SemaphoreType.DMA,
        pltpu.SemaphoreType.DMA,
        pltpu.SemaphoreType.DMA,
        pltpu.SemaphoreType.DMA,
    ],
)
def _spmm_fsplit(u0_hbm, u1_hbm, src_hbm, dst_hbm, a0_out, a1_out,
                 acc, sv0, dv0, sv1, dv1, rows_v,
                 isv0, idv0, isv1, idv1, gsem):
    c = lax.axis_index("c")
    s = lax.axis_index("s")
    r0 = s * ROWS_PER_TILE

    nblk = E_PAD // (NS * BLK)
    e0 = s * nblk * BLK

    pltpu.async_copy(src_hbm.at[pl.ds(e0, BLK)], sv0, isv0)
    pltpu.async_copy(dst_hbm.at[pl.ds(e0, BLK)], dv0, idv0)
    pltpu.async_copy(src_hbm.at[pl.ds(e0 + BLK, BLK)], sv1, isv1)
    pltpu.async_copy(dst_hbm.at[pl.ds(e0 + BLK, BLK)], dv1, idv1)

    @pl.when(c == 0)
    def _():
        pltpu.sync_copy(u0_hbm.at[pl.ds(r0, ROWS_PER_TILE)],
                        acc.at[pl.ds(r0, ROWS_PER_TILE)])

    @pl.when(c == 1)
    def _():
        pltpu.sync_copy(u1_hbm.at[pl.ds(r0, ROWS_PER_TILE)],
                        acc.at[pl.ds(r0, ROWS_PER_TILE)])

    plsc.subcore_barrier()

    def half(sv, dv, isv, idv, base, nxt, has_nxt):
        pltpu.make_async_copy(src_hbm.at[pl.ds(base, BLK)], sv, isv).wait()
        pltpu.make_async_copy(dst_hbm.at[pl.ds(base, BLK)], dv, idv).wait()

        @pl.when(c == 0)
        def _():
            pltpu.async_copy(u0_hbm.at[sv], rows_v, gsem).wait()

        @pl.when(c == 1)
        def _():
            pltpu.async_copy(u1_hbm.at[sv], rows_v, gsem).wait()

        pltpu.sync_copy(rows_v, acc.at[dv], add=True)

        @pl.when(has_nxt)
        def _():
            pltpu.async_copy(src_hbm.at[pl.ds(nxt, BLK)], sv, isv)
            pltpu.async_copy(dst_hbm.at[pl.ds(nxt, BLK)], dv, idv)

    def pair(j, _):
        b = e0 + 2 * j * BLK
        half(sv0, dv0, isv0, idv0, b, b + 2 * BLK, 2 * j + 2 < nblk)
        half(sv1, dv1, isv1, idv1, b + BLK, b + 3 * BLK, 2 * j + 3 < nblk)
        return 0

    lax.fori_loop(0, nblk // 2, pair, 0, unroll=False)
    if nblk % 2:
        b_t = e0 + (nblk - 1) * BLK
        half(sv0, dv0, isv0, idv0, b_t, b_t, jnp.bool_(False))
    plsc.subcore_barrier()

    @pl.when(c == 0)
    def _():
        pltpu.sync_copy(acc.at[pl.ds(r0, ROWS_PER_TILE)],
                        a0_out.at[pl.ds(r0, ROWS_PER_TILE)])

    @pl.when(c == 1)
    def _():
        pltpu.sync_copy(acc.at[pl.ds(r0, ROWS_PER_TILE)],
                        a1_out.at[pl.ds(r0, ROWS_PER_TILE)])


@functools.partial(
    pl.kernel,
    out_type=(_f32((N_PAD, 128)), _f32((N_PAD, 128))),
    mesh=_mesh,
    scratch_types=[
        pltpu.VMEM_SHARED((N_PAD, 128), jnp.float32),
        pltpu.VMEM((BLK,), jnp.int32),
        pltpu.VMEM((BLK,), jnp.int32),
        pltpu.VMEM((BLK,), jnp.int32),
        pltpu.VMEM((BLK,), jnp.int32),
        pltpu.VMEM((BLK, 128), jnp.float32),
        pltpu.SemaphoreType.DMA,
        pltpu.SemaphoreType.DMA,
        pltpu.SemaphoreType.DMA,
        pltpu.SemaphoreType.DMA,
        pltpu.SemaphoreType.DMA,
    ],
)
def _spmm_esplit(u_hbm, src_hbm, dst_hbm, aa_out, ab_out,
                 acc, sv0, dv0, sv1, dv1, rows_v,
                 isv0, idv0, isv1, idv1, gsem):
    c = lax.axis_index("c")
    s = lax.axis_index("s")
    r0 = s * ROWS_PER_TILE

    nblk = E_PAD // (NC * NS * BLK)
    e0 = (c * NS + s) * nblk * BLK

    pltpu.async_copy(src_hbm.at[pl.ds(e0, BLK)], sv0, isv0)
    pltpu.async_copy(dst_hbm.at[pl.ds(e0, BLK)], dv0, idv0)
    pltpu.async_copy(src_hbm.at[pl.ds(e0 + BLK, BLK)], sv1, isv1)
    pltpu.async_copy(dst_hbm.at[pl.ds(e0 + BLK, BLK)], dv1, idv1)

    pltpu.sync_copy(u_hbm.at[pl.ds(r0, ROWS_PER_TILE)],
                    acc.at[pl.ds(r0, ROWS_PER_TILE)])
    plsc.subcore_barrier()

    def half(sv, dv, isv, idv, base, nxt, has_nxt):
        pltpu.make_async_copy(src_hbm.at[pl.ds(base, BLK)], sv, isv).wait()
        pltpu.make_async_copy(dst_hbm.at[pl.ds(base, BLK)], dv, idv).wait()
        pltpu.async_copy(u_hbm.at[sv], rows_v, gsem).wait()
        pltpu.sync_copy(rows_v, acc.at[dv], add=True)

        @pl.when(has_nxt)
        def _():
            pltpu.async_copy(src_hbm.at[pl.ds(nxt, BLK)], sv, isv)
            pltpu.async_copy(dst_hbm.at[pl.ds(nxt, BLK)], dv, idv)

    def pair(j, _):
        b = e0 + 2 * j * BLK
        half(sv0, dv0, isv0, idv0, b, b + 2 * BLK, 2 * j + 2 < nblk)
        half(sv1, dv1, isv1, idv1, b + BLK, b + 3 * BLK, 2 * j + 3 < nblk)
        return 0

    lax.fori_loop(0, nblk // 2, pair, 0, unroll=False)
    if nblk % 2:
        b_t = e0 + (nblk - 1) * BLK
        half(sv0, dv0, isv0, idv0, b_t, b_t, jnp.bool_(False))
    plsc.subcore_barrier()

    @pl.when(c == 0)
    def _():
        pltpu.sync_copy(acc.at[pl.ds(r0, ROWS_PER_TILE)],
                        aa_out.at[pl.ds(r0, ROWS_PER_TILE)])

    @pl.when(c == 1)
    def _():
        pltpu.sync_copy(acc.at[pl.ds(r0, ROWS_PER_TILE)],
                        ab_out.at[pl.ds(r0, ROWS_PER_TILE)])


def _row_spec(width):
    return pl.BlockSpec((BM, width), lambda i: (i, 0))


def _full_spec(shape):
    return pl.BlockSpec(shape, lambda i: tuple(0 for _ in shape))


def _l1_body(x_ref, w_ref, g_ref, v_ref, deg0_ref, deg1_ref,
             u0_ref, u1_ref, dinv_ref):
    i = pl.program_id(0)
    rows = i * BM + lax.broadcasted_iota(jnp.int32, (BM, 1), 0)
    deg = deg0_ref[:, 0:1] + deg1_ref[:, 0:1] + 1.0
    dinv = jnp.where(rows < N, lax.rsqrt(deg), 0.0)
    dinv_ref[...] = dinv
    s = g_ref[...] * lax.rsqrt(v_ref[...] + EPS)
    h = jnp.dot(x_ref[...], w_ref[...], preferred_element_type=jnp.float32)
    u = h * s * dinv
    u0_ref[...] = u[:, :128]
    u1_ref[...] = u[:, 128:]


def _mid_body(a0_ref, a1_ref, dinv_ref, b_ref, g_ref, beta_ref, m_ref, v_ref,
              gn_ref, vn_ref, w_ref, u0_ref, u1_ref):
    sp = g_ref[...] * lax.rsqrt(v_ref[...] + EPS)
    bias = b_ref[...] * sp + beta_ref[...] - m_ref[...] * sp
    dinv = dinv_ref[...]
    acc = jnp.concatenate([a0_ref[...], a1_ref[...]], axis=1)
    t = jnp.maximum(acc * dinv + bias, 0.0)
    sn = gn_ref[...] * lax.rsqrt(vn_ref[...] + EPS)
    u = jnp.dot(t, w_ref[...], preferred_element_type=jnp.float32) * sn * dinv
    u0_ref[...] = u[:, :128]
    u1_ref[...] = u[:, 128:]


def _l3_body(a0_ref, a1_ref, dinv_ref, b_ref, g_ref, beta_ref, m_ref, v_ref,
             w_ref, u_ref):
    sp = g_ref[...] * lax.rsqrt(v_ref[...] + EPS)
    bias = b_ref[...] * sp + beta_ref[...] - m_ref[...] * sp
    dinv = dinv_ref[...]
    acc = jnp.concatenate([a0_ref[...], a1_ref[...]], axis=1)
    t = jnp.maximum(acc * dinv + bias, 0.0)
    u_ref[...] = jnp.dot(t, w_ref[...],
                         preferred_element_type=jnp.float32) * dinv


def _fin_body(aa_ref, ab_ref, u_ref, dinv_ref, b_ref, o_ref):
    acc = aa_ref[...] + ab_ref[...] - u_ref[...]
    o_ref[...] = acc * dinv_ref[...] + b_ref[...]


_GRID = (N_PAD // BM,)

_l1_call = pl.pallas_call(
    _l1_body,
    grid=_GRID,
    in_specs=[_row_spec(128), _full_spec((128, 256)), _full_spec((1, 256)),
              _full_spec((1, 256)), _row_spec(16), _row_spec(16)],
    out_specs=[_row_spec(128), _row_spec(128), _row_spec(1)],
    out_shape=[_f32((N_PAD, 128)), _f32((N_PAD, 128)), _f32((N_PAD, 1))],
)

_mid_call = pl.pallas_call(
    _mid_body,
    grid=_GRID,
    in_specs=[_row_spec(128), _row_spec(128), _row_spec(1)]
             + [_full_spec((1, 256))] * 7
             + [_full_spec((256, 256))],
    out_specs=[_row_spec(128), _row_spec(128)],
    out_shape=[_f32((N_PAD, 128)), _f32((N_PAD, 128))],
)

_l3_call = pl.pallas_call(
    _l3_body,
    grid=_GRID,
    in_specs=[_row_spec(128), _row_spec(128), _row_spec(1)]
             + [_full_spec((1, 256))] * 5
             + [_full_spec((256, 128))],
    out_specs=_row_spec(128),
    out_shape=_f32((N_PAD, 128)),
)

_fin_call = pl.pallas_call(
    _fin_body,
    grid=_GRID,
    in_specs=[_row_spec(128), _row_spec(128), _row_spec(128), _row_spec(1),
              _full_spec((1, 128))],
    out_specs=_row_spec(128),
    out_shape=_f32((N_PAD, 128)),
)


@jax.jit
def kernel(x, edge_index, W1, b1, W2, b2, W3, b3,
           g1, beta1, m1, v1, g2, beta2, m2, v2):
    src = edge_index[0].astype(jnp.int32)
    dst = edge_index[1].astype(jnp.int32)
    pad = jnp.full((E_PAD - E,), DUMMY, jnp.int32)
    src_p = jnp.concatenate([src, pad])
    dst_p = jnp.concatenate([dst, pad])
    x_p = jnp.zeros((N_PAD, 128), jnp.float32).at[:N].set(x)

    row = lambda a: a.reshape(1, -1)

    deg0, deg1 = _deg_kernel(dst_p)

    u1_0, u1_1, dinv = _l1_call(x_p, W1, row(g1), row(v1), deg0, deg1)
    a1_0, a1_1 = _spmm_fsplit(u1_0, u1_1, src_p, dst_p)

    u2_0, u2_1 = _mid_call(a1_0, a1_1, dinv, row(b1), row(g1), row(beta1),
                           row(m1), row(v1), row(g2), row(v2), W2)
    a2_0, a2_1 = _spmm_fsplit(u2_0, u2_1, src_p, dst_p)

    u3 = _l3_call(a2_0, a2_1, dinv, row(b2), row(g2), row(beta2),
                  row(m2), row(v2), W3)
    a3a, a3b = _spmm_esplit(u3, src_p, dst_p)

    out = _fin_call(a3a, a3b, u3, dinv, row(b3))
    return out[:N]
